# Initial kernel scaffold; baseline (speedup 1.0000x reference)
import jax
import jax.numpy as jnp
from jax.experimental import pallas as pl


def kernel(x, embeddings):
    raise NotImplementedError("write your pallas kernel here")


# speedup vs baseline: 1.6524x; 1.6524x over previous
import functools

import jax
import jax.numpy as jnp
from jax import lax
from jax.experimental import pallas as pl
from jax.experimental.pallas import tpu as pltpu
from jax.experimental.pallas import tpu_sc as plsc

_K = 8192
_D = 256
_MBLK = 256


def _argmin_body(x_ref, e_ref, idx_ref, coln_ref):
    @pl.when(pl.program_id(0) == 0)
    def _():
        coln_ref[...] = jnp.sum(jnp.square(e_ref[...]), axis=0, keepdims=True)

    xb = x_ref[...]
    rown = jnp.sum(jnp.square(xb), axis=-1, keepdims=True)
    mm = jnp.dot(xb, e_ref[...])
    dis = (rown + coln_ref[...]) - 2.0 * mm
    tmin = jnp.min(dis, axis=1, keepdims=True)
    cols = lax.broadcasted_iota(jnp.int32, dis.shape, 1)
    idx_ref[...] = jnp.min(
        jnp.where(dis == tmin, cols, jnp.int32(2**31 - 1)),
        axis=1, keepdims=True)


def _argmin_call(xf, emb):
    n = xf.shape[0]
    return pl.pallas_call(
        _argmin_body,
        grid=(n // _MBLK,),
        in_specs=[
            pl.BlockSpec((_MBLK, _D), lambda m: (m, 0)),
            pl.BlockSpec((_D, _K), lambda m: (0, 0)),
        ],
        out_specs=pl.BlockSpec((_MBLK, 1), lambda m: (m, 0)),
        out_shape=jax.ShapeDtypeStruct((n, 1), jnp.int32),
        scratch_shapes=[pltpu.VMEM((1, _K), jnp.float32)],
    )(xf, emb)


def _gather_call(table, idx):
    info = plsc.get_sparse_core_info()
    nc, ns = info.num_cores, info.num_subcores
    nw = nc * ns
    b = idx.shape[0]
    ch = 128
    n_ch = b // (nw * ch)
    idx2 = idx.reshape(nw * n_ch, ch)
    mesh = plsc.VectorSubcoreMesh(core_axis_name="c", subcore_axis_name="s")

    @functools.partial(
        pl.kernel, mesh=mesh,
        out_type=jax.ShapeDtypeStruct((b, _D), jnp.float32),
        scratch_types=[
            pltpu.VMEM((n_ch, ch), jnp.int32),
            pltpu.VMEM((ch, _D), jnp.float32),
            pltpu.SemaphoreType.DMA,
        ],
    )
    def k(table_hbm, idx_hbm, out_hbm, idx_v, rows_v, sem):
        wid = lax.axis_index("s") * nc + lax.axis_index("c")
        pltpu.sync_copy(idx_hbm.at[pl.ds(wid * n_ch, n_ch)], idx_v)
        for c in range(n_ch):
            pltpu.async_copy(table_hbm.at[idx_v.at[c]], rows_v, sem).wait()
            pltpu.sync_copy(rows_v, out_hbm.at[pl.ds((wid * n_ch + c) * ch, ch)])

    return k(table, idx2)


def kernel(x, embeddings):
    xf = x.reshape(-1, x.shape[-1])
    idx = _argmin_call(xf, embeddings)[:, 0]
    quant = _gather_call(embeddings.T, idx)
    return quant.reshape(x.shape)


# speedup vs baseline: 1.6613x; 1.0054x over previous
import functools

import jax
import jax.numpy as jnp
from jax import lax
from jax.experimental import pallas as pl
from jax.experimental.pallas import tpu as pltpu
from jax.experimental.pallas import tpu_sc as plsc

_K = 8192
_D = 256
_MBLK = 256


def _argmin_body(x_ref, e_ref, idx_ref, coln_ref, iota_ref):
    @pl.when(pl.program_id(0) == 0)
    def _():
        coln_ref[...] = jnp.sum(jnp.square(e_ref[...]), axis=0, keepdims=True)
        iota_ref[...] = lax.broadcasted_iota(
            jnp.int32, (1, _K), 1).astype(jnp.float32)

    xb = x_ref[...]
    rown = jnp.sum(jnp.square(xb), axis=-1, keepdims=True)
    mm2 = jnp.dot(xb + xb, e_ref[...])
    dis = (rown + coln_ref[...]) - mm2
    tmin = jnp.min(dis, axis=1, keepdims=True)
    idxf = jnp.min(jnp.where(dis == tmin, iota_ref[...], jnp.inf),
                   axis=1, keepdims=True)
    idx_ref[...] = idxf.astype(jnp.int32)


def _argmin_call(xf, emb):
    n = xf.shape[0]
    return pl.pallas_call(
        _argmin_body,
        grid=(n // _MBLK,),
        in_specs=[
            pl.BlockSpec((_MBLK, _D), lambda m: (m, 0)),
            pl.BlockSpec((_D, _K), lambda m: (0, 0)),
        ],
        out_specs=pl.BlockSpec((_MBLK, 1), lambda m: (m, 0)),
        out_shape=jax.ShapeDtypeStruct((n, 1), jnp.int32),
        scratch_shapes=[pltpu.VMEM((1, _K), jnp.float32),
                        pltpu.VMEM((1, _K), jnp.float32)],
    )(xf, emb)


def _gather_call(table, idx):
    info = plsc.get_sparse_core_info()
    nc, ns = info.num_cores, info.num_subcores
    nw = nc * ns
    b = idx.shape[0]
    ch = 128
    n_ch = b // (nw * ch)
    idx2 = idx.reshape(nw * n_ch, ch)
    mesh = plsc.VectorSubcoreMesh(core_axis_name="c", subcore_axis_name="s")

    @functools.partial(
        pl.kernel, mesh=mesh,
        out_type=jax.ShapeDtypeStruct((b, _D), jnp.float32),
        scratch_types=[
            pltpu.VMEM((n_ch, ch), jnp.int32),
            pltpu.VMEM((ch, _D), jnp.float32),
            pltpu.SemaphoreType.DMA,
        ],
    )
    def k(table_hbm, idx_hbm, out_hbm, idx_v, rows_v, sem):
        wid = lax.axis_index("s") * nc + lax.axis_index("c")
        pltpu.sync_copy(idx_hbm.at[pl.ds(wid * n_ch, n_ch)], idx_v)
        for c in range(n_ch):
            pltpu.async_copy(table_hbm.at[idx_v.at[c]], rows_v, sem).wait()
            pltpu.sync_copy(rows_v, out_hbm.at[pl.ds((wid * n_ch + c) * ch, ch)])

    return k(table, idx2)


def kernel(x, embeddings):
    xf = x.reshape(-1, x.shape[-1])
    idx = _argmin_call(xf, embeddings)[:, 0]
    quant = _gather_call(embeddings.T, idx)
    return quant.reshape(x.shape)


# speedup vs baseline: 2.0754x; 1.2493x over previous
import functools

import jax
import jax.numpy as jnp
from jax import lax
from jax.experimental import pallas as pl
from jax.experimental.pallas import tpu as pltpu
from jax.experimental.pallas import tpu_sc as plsc

_K = 8192
_D = 256
_MBLK = 256


_NCHUNK = 128


def _argmin_body(x_ref, e_ref, idx_ref, coln_ref):
    @pl.when(pl.program_id(0) == 0)
    def _():
        coln_ref[...] = jnp.sum(jnp.square(e_ref[...]), axis=0, keepdims=True)

    xb = x_ref[...]
    rown = jnp.sum(jnp.square(xb), axis=-1, keepdims=True)
    mm2 = jnp.dot(xb + xb, e_ref[...])
    accv = jnp.full((_MBLK, _NCHUNK), jnp.inf, jnp.float32)
    acct = jnp.zeros((_MBLK, _NCHUNK), jnp.int32)
    for j in range(_K // _NCHUNK):
        lo, hi = j * _NCHUNK, (j + 1) * _NCHUNK
        d = (rown + coln_ref[:, lo:hi]) - mm2[:, lo:hi]
        upd = d < accv
        accv = jnp.where(upd, d, accv)
        acct = jnp.where(upd, j, acct)
    lane = lax.broadcasted_iota(jnp.int32, (_MBLK, _NCHUNK), 1)
    idx128 = acct * _NCHUNK + lane
    tmin = jnp.min(accv, axis=1, keepdims=True)
    idx_ref[...] = jnp.min(
        jnp.where(accv == tmin, idx128, jnp.int32(2**31 - 1)),
        axis=1, keepdims=True)


def _argmin_call(xf, emb):
    n = xf.shape[0]
    return pl.pallas_call(
        _argmin_body,
        grid=(n // _MBLK,),
        in_specs=[
            pl.BlockSpec((_MBLK, _D), lambda m: (m, 0)),
            pl.BlockSpec((_D, _K), lambda m: (0, 0)),
        ],
        out_specs=pl.BlockSpec((_MBLK, 1), lambda m: (m, 0)),
        out_shape=jax.ShapeDtypeStruct((n, 1), jnp.int32),
        scratch_shapes=[pltpu.VMEM((1, _K), jnp.float32)],
    )(xf, emb)


def _gather_call(table, idx):
    info = plsc.get_sparse_core_info()
    nc, ns = info.num_cores, info.num_subcores
    nw = nc * ns
    b = idx.shape[0]
    ch = 128
    n_ch = b // (nw * ch)
    idx2 = idx.reshape(nw * n_ch, ch)
    mesh = plsc.VectorSubcoreMesh(core_axis_name="c", subcore_axis_name="s")

    @functools.partial(
        pl.kernel, mesh=mesh,
        out_type=jax.ShapeDtypeStruct((b, _D), jnp.float32),
        scratch_types=[
            pltpu.VMEM((n_ch, ch), jnp.int32),
            pltpu.VMEM((ch, _D), jnp.float32),
            pltpu.SemaphoreType.DMA,
        ],
    )
    def k(table_hbm, idx_hbm, out_hbm, idx_v, rows_v, sem):
        wid = lax.axis_index("s") * nc + lax.axis_index("c")
        pltpu.sync_copy(idx_hbm.at[pl.ds(wid * n_ch, n_ch)], idx_v)
        for c in range(n_ch):
            pltpu.async_copy(table_hbm.at[idx_v.at[c]], rows_v, sem).wait()
            pltpu.sync_copy(rows_v, out_hbm.at[pl.ds((wid * n_ch + c) * ch, ch)])

    return k(table, idx2)


def kernel(x, embeddings):
    xf = x.reshape(-1, x.shape[-1])
    idx = _argmin_call(xf, embeddings)[:, 0]
    quant = _gather_call(embeddings.T, idx)
    return quant.reshape(x.shape)


# speedup vs baseline: 2.1628x; 1.0421x over previous
import functools

import jax
import jax.numpy as jnp
from jax import lax
from jax.experimental import pallas as pl
from jax.experimental.pallas import tpu as pltpu
from jax.experimental.pallas import tpu_sc as plsc

_K = 8192
_D = 256
_MBLK = 512


_NCHUNK = 128


def _argmin_body(x_ref, e_ref, idx_ref, coln_ref):
    @pl.when(pl.program_id(0) == 0)
    def _():
        coln_ref[...] = jnp.sum(jnp.square(e_ref[...]), axis=0, keepdims=True)

    xb = x_ref[...]
    rown = jnp.sum(jnp.square(xb), axis=-1, keepdims=True)
    mm2 = jnp.dot(xb + xb, e_ref[...])
    accv = jnp.full((_MBLK, _NCHUNK), jnp.inf, jnp.float32)
    acct = jnp.zeros((_MBLK, _NCHUNK), jnp.int32)
    for j in range(_K // _NCHUNK):
        lo, hi = j * _NCHUNK, (j + 1) * _NCHUNK
        d = (rown + coln_ref[:, lo:hi]) - mm2[:, lo:hi]
        upd = d < accv
        accv = jnp.where(upd, d, accv)
        acct = jnp.where(upd, j, acct)
    lane = lax.broadcasted_iota(jnp.int32, (_MBLK, _NCHUNK), 1)
    idx128 = acct * _NCHUNK + lane
    tmin = jnp.min(accv, axis=1, keepdims=True)
    idx_ref[...] = jnp.min(
        jnp.where(accv == tmin, idx128, jnp.int32(2**31 - 1)),
        axis=1, keepdims=True)


def _argmin_call(xf, emb):
    n = xf.shape[0]
    return pl.pallas_call(
        _argmin_body,
        grid=(n // _MBLK,),
        in_specs=[
            pl.BlockSpec((_MBLK, _D), lambda m: (m, 0)),
            pl.BlockSpec((_D, _K), lambda m: (0, 0)),
        ],
        out_specs=pl.BlockSpec((_MBLK, 1), lambda m: (m, 0)),
        out_shape=jax.ShapeDtypeStruct((n, 1), jnp.int32),
        scratch_shapes=[pltpu.VMEM((1, _K), jnp.float32)],
    )(xf, emb)


def _gather_call(table, idx):
    info = plsc.get_sparse_core_info()
    nc, ns = info.num_cores, info.num_subcores
    nw = nc * ns
    b = idx.shape[0]
    ch = 128
    n_ch = b // (nw * ch)
    idx2 = idx.reshape(nw * n_ch, ch)
    mesh = plsc.VectorSubcoreMesh(core_axis_name="c", subcore_axis_name="s")

    @functools.partial(
        pl.kernel, mesh=mesh,
        out_type=jax.ShapeDtypeStruct((b, _D), jnp.float32),
        scratch_types=[
            pltpu.VMEM((n_ch, ch), jnp.int32),
            pltpu.VMEM((ch, _D), jnp.float32),
            pltpu.SemaphoreType.DMA,
        ],
    )
    def k(table_hbm, idx_hbm, out_hbm, idx_v, rows_v, sem):
        wid = lax.axis_index("s") * nc + lax.axis_index("c")
        pltpu.sync_copy(idx_hbm.at[pl.ds(wid * n_ch, n_ch)], idx_v)
        for c in range(n_ch):
            pltpu.async_copy(table_hbm.at[idx_v.at[c]], rows_v, sem).wait()
            pltpu.sync_copy(rows_v, out_hbm.at[pl.ds((wid * n_ch + c) * ch, ch)])

    return k(table, idx2)


def kernel(x, embeddings):
    xf = x.reshape(-1, x.shape[-1])
    idx = _argmin_call(xf, embeddings)[:, 0]
    quant = _gather_call(embeddings.T, idx)
    return quant.reshape(x.shape)
